# baseline (speedup 1.0000x reference)
import functools

import jax
import jax.numpy as jnp
from jax import lax
from jax.experimental import pallas as pl
from jax.experimental.pallas import tpu as pltpu
from jax.experimental.pallas import tpu_sc as plsc

T, B, D = 2048, 4, 1024
NC, NS, L = 2, 16, 16
NW = NC * NS
TPW = T // NW
CHUNK = 8
NCHUNK = TPW // CHUNK
VECS = D // L


def _sc_body(x_hbm, pos_hbm, out_hbm, x_v, pos_v, in_sems, out_sems):
    wid = lax.axis_index("s") * NC + lax.axis_index("c")
    base = wid * TPW

    def start_in(c, slot):
        t0 = base + c * CHUNK
        pltpu.async_copy(x_hbm.at[pl.ds(t0, CHUNK)], x_v.at[slot], in_sems.at[slot])
        pltpu.async_copy(pos_hbm.at[pl.ds(t0, CHUNK)], pos_v.at[slot], in_sems.at[slot])

    def wait_in(slot):
        pltpu.make_async_copy(x_hbm.at[pl.ds(0, CHUNK)], x_v.at[slot], in_sems.at[slot]).wait()
        pltpu.make_async_copy(pos_hbm.at[pl.ds(0, CHUNK)], pos_v.at[slot], in_sems.at[slot]).wait()

    def start_out(c, slot):
        t0 = base + c * CHUNK
        pltpu.async_copy(x_v.at[slot], out_hbm.at[pl.ds(t0, CHUNK)], out_sems.at[slot])

    def wait_out(slot):
        pltpu.make_async_copy(x_v.at[slot], out_hbm.at[pl.ds(0, CHUNK)], out_sems.at[slot]).wait()

    def compute(slot):
        def row_body(t, carry):
            for j in range(VECS):
                p = pos_v[slot, t, pl.ds(j * L, L)]
                for b in range(B):
                    plsc.addupdate(x_v.at[slot, t, b, pl.ds(j * L, L)], p)
            return carry

        lax.fori_loop(0, CHUNK, row_body, 0)

    start_in(0, 0)
    for c in range(NCHUNK):
        slot = c % 2
        if c + 1 < NCHUNK:
            if c >= 1:
                wait_out(1 - slot)
            start_in(c + 1, 1 - slot)
        wait_in(slot)
        compute(slot)
        start_out(c, slot)
    wait_out(0)
    wait_out(1)


def kernel(x, pos_table):
    mesh = plsc.VectorSubcoreMesh(core_axis_name="c", subcore_axis_name="s")
    k = functools.partial(
        pl.kernel,
        mesh=mesh,
        out_type=jax.ShapeDtypeStruct((T, B, D), jnp.float32),
        scratch_types=[
            pltpu.VMEM((2, CHUNK, B, D), jnp.float32),
            pltpu.VMEM((2, CHUNK, D), jnp.float32),
            pltpu.SemaphoreType.DMA((2,)),
            pltpu.SemaphoreType.DMA((2,)),
        ],
    )(_sc_body)
    return k(x, pos_table)
